# baseline (speedup 1.0000x reference)
import functools
import math

import jax
import jax.numpy as jnp
import numpy as np
from jax import lax
from jax.experimental import pallas as pl
from jax.experimental.pallas import tpu as pltpu
from jax.experimental.pallas import tpu_sc as plsc

B = 2
NQ = 10000
E = 256
H = 8
D = 32
P = 4
GH = 100
GW = 100
HW = GH * GW
ITEMS = B * NQ * H
K = 4 * P

NTILES = 32
IPT = ITEMS // NTILES
C = 40
NCHUNK = IPT // C
GSUB = (C * K) // 128

BQ = 2000
NBLK = (B * NQ) // BQ


def _proj_kernel(q_ref, v_ref, rp_ref, wv_ref, bv_ref, wsox_ref, bsox_ref,
                 wsoy_ref, bsoy_ref, waw_ref, baw_ref, r_ref, s_ref, c_ref,
                 vout_ref, idx_ref, w_ref):
    f32 = jnp.float32
    q = q_ref[...]
    vout_ref[...] = (jnp.dot(v_ref[...], wv_ref[...],
                             preferred_element_type=f32)
                     + bv_ref[...]).astype(jnp.bfloat16)
    sox = jnp.dot(q, wsox_ref[...], preferred_element_type=f32) + bsox_ref[...]
    soy = jnp.dot(q, wsoy_ref[...], preferred_element_type=f32) + bsoy_ref[...]
    awl = jnp.dot(q, waw_ref[...], preferred_element_type=f32) + baw_ref[...]
    e = jnp.exp(awl - jnp.max(awl, axis=-1, keepdims=True))
    awn = e / jnp.dot(e, s_ref[...], preferred_element_type=f32)
    x = (rp_ref[:, 0:1] + sox / f32(GW)) * f32(GW) - f32(0.5)
    y = (rp_ref[:, 1:2] + soy / f32(GH)) * f32(GH) - f32(0.5)
    x0 = jnp.floor(x)
    y0 = jnp.floor(y)
    lx = x - x0
    ly = y - y0
    r = r_ref[...]
    X0 = jnp.dot(x0, r, preferred_element_type=f32)
    Y0 = jnp.dot(y0, r, preferred_element_type=f32)
    LX = jnp.dot(lx, r, preferred_element_type=f32)
    LY = jnp.dot(ly, r, preferred_element_type=f32)
    AW = jnp.dot(awn, r, preferred_element_type=f32)
    dx = c_ref[0:1, :]
    dy = c_ref[1:2, :]
    hl = c_ref[2:3, :]
    xi = X0 + dx
    yi = Y0 + dy
    fx = dx * LX + (f32(1.0) - dx) * (f32(1.0) - LX)
    fy = dy * LY + (f32(1.0) - dy) * (f32(1.0) - LY)
    m = ((xi >= f32(0.0)) & (xi <= f32(GW - 1)) & (yi >= f32(0.0))
         & (yi <= f32(GH - 1))).astype(f32)
    cell = (jnp.clip(yi, f32(0.0), f32(GH - 1)) * f32(GW)
            + jnp.clip(xi, f32(0.0), f32(GW - 1)))
    boff = lax.convert_element_type(
        (pl.program_id(0) // (NQ // BQ)) * (HW * H), f32)
    idx_ref[...] = (cell * f32(H) + (boff + hl)).astype(jnp.int32)
    w_ref[...] = AW * fx * fy * m


def _out_kernel(s_ref, q_ref, wo_ref, bo_ref, out_ref):
    out_ref[...] = (jnp.dot(s_ref[...], wo_ref[...],
                            preferred_element_type=jnp.float32)
                    + bo_ref[...] + q_ref[...])


QC = 5
QPT = (B * NQ) // NTILES


def _bcast_lane(v, k):
    return lax.gather(
        v, jnp.full((16, 1), k, jnp.int32),
        lax.GatherDimensionNumbers(offset_dims=(), collapsed_slice_dims=(0,),
                                   start_index_map=(0,)),
        slice_sizes=(1,), mode=lax.GatherScatterMode.PROMISE_IN_BOUNDS)


NCH = QPT // QC


def _sc_gather_kernel(table, idxp, wp, out,
                      idx_v0, idx_v1, w_v0, w_v1, rows_v0, rows_v1,
                      out_v0, out_v1,
                      sem_in0, sem_in1, sem_g0, sem_g1, sem_out0, sem_out1):
    wid = lax.axis_index("s") * 2 + lax.axis_index("c")
    base = wid * QPT
    bufs = ((idx_v0, w_v0, rows_v0, out_v0, sem_in0, sem_g0, sem_out0),
            (idx_v1, w_v1, rows_v1, out_v1, sem_in1, sem_g1, sem_out1))

    def in_cps(g, buf):
        idx_v, w_v, _, _, sem_in, _, _ = bufs[buf]
        q0 = base + g * QC
        return (pltpu.make_async_copy(idxp.at[pl.ds(q0, QC), :], idx_v, sem_in),
                pltpu.make_async_copy(wp.at[pl.ds(q0, QC), :], w_v, sem_in))

    def g_cps(buf):
        idx_v, _, rows_v, _, _, sem_g, _ = bufs[buf]
        return [pltpu.make_async_copy(table.at[idx_v.at[j, :]],
                                      rows_v.at[pl.ds(j * 128, 128), :], sem_g)
                for j in range(QC)]

    def out_cp(g, buf):
        _, _, _, out_v, _, _, sem_out = bufs[buf]
        q0 = base + g * QC
        return pltpu.make_async_copy(out_v, out.at[pl.ds(q0, QC), :], sem_out)

    def compute(buf):
        _, w_v, rows_v, out_v, _, _, _ = bufs[buf]

        def query(ql, c2):
            for h in range(H):
                acc0 = jnp.zeros((16,), jnp.float32)
                acc1 = jnp.zeros((16,), jnp.float32)
                wv = w_v[ql, pl.ds(h * K, K)]
                for k in range(K):
                    r = ql * 128 + h * K + k
                    lo, hi = plsc.unpack(rows_v[r, :],
                                         format=plsc.PackFormat.INTERLEAVED,
                                         preferred_element_type=jnp.float32)
                    wk = _bcast_lane(wv, k)
                    acc0 = acc0 + wk * lo
                    acc1 = acc1 + wk * hi
                out_v[ql, pl.ds(h * D, 16)] = acc0
                out_v[ql, pl.ds(h * D + 16, 16)] = acc1
            return c2

        lax.fori_loop(0, QC, query, 0, unroll=False)

    def step(g, buf, i):
        for cp in g_cps(buf):
            cp.wait()
        for cp in in_cps(g + 1, 1 - buf):
            cp.wait()
        for cp in g_cps(1 - buf):
            cp.start()
        @pl.when(i >= 1)
        def _drain():
            out_cp(g - 2, buf).wait()

        compute(buf)
        out_cp(g, buf).start()

        @pl.when(g + 2 < NCH)
        def _prefetch():
            for cp in in_cps(g + 2, buf):
                cp.start()

    for cp in in_cps(0, 0) + in_cps(1, 1):
        cp.start()
    for cp in in_cps(0, 0):
        cp.wait()
    for cp in g_cps(0):
        cp.start()

    def body(i, carry):
        step(2 * i, 0, i)
        step(2 * i + 1, 1, i)
        return carry

    lax.fori_loop(0, (NCH - 1) // 2, body, 0, unroll=False)

    g_last = NCH - 1
    for cp in g_cps(0):
        cp.wait()
    out_cp(g_last - 2, 0).wait()
    compute(0)
    out_cp(g_last, 0).start()
    out_cp(g_last - 1, 1).wait()
    out_cp(g_last, 0).wait()


def _build_idx_w(so, aw_logits, reference_points):
    so = so.reshape(B, NQ, H, P, 2)
    aw = jax.nn.softmax(aw_logits.reshape(B, NQ, H, P), axis=-1)
    rp = reference_points.reshape(B, NQ, 1, 1, 2)
    loc = rp + so / jnp.array([GW, GH], jnp.float32)
    x = loc[..., 0] * GW - 0.5
    y = loc[..., 1] * GH - 0.5
    x0 = jnp.floor(x)
    y0 = jnp.floor(y)
    lx = x - x0
    ly = y - y0
    x0i = x0.astype(jnp.int32)
    y0i = y0.astype(jnp.int32)
    b_off = (jnp.arange(B, dtype=jnp.int32) * HW).reshape(B, 1, 1, 1)
    h_off = jnp.arange(H, dtype=jnp.int32).reshape(1, 1, H, 1)
    idxs, ws = [], []
    for dy, dx, cw in ((0, 0, (1 - ly) * (1 - lx)), (0, 1, (1 - ly) * lx),
                       (1, 0, ly * (1 - lx)), (1, 1, ly * lx)):
        yi = y0i + dy
        xi = x0i + dx
        m = (yi >= 0) & (yi < GH) & (xi >= 0) & (xi < GW)
        cell = jnp.clip(yi, 0, GH - 1) * GW + jnp.clip(xi, 0, GW - 1)
        idxs.append((b_off + cell) * H + h_off)
        ws.append(aw * cw * m.astype(jnp.float32))
    idx = jnp.stack(idxs, axis=-1)
    w = jnp.stack(ws, axis=-1)
    return idx.reshape(-1), w.reshape(-1)


def kernel(query, value, reference_points, spatial_shapes, W_so, b_so, W_aw,
           b_aw, W_v, b_v, W_o, b_o):
    q2 = query.reshape(B * NQ, E)
    v2 = value.reshape(B * NQ, E)
    rp2 = reference_points.reshape(B * NQ, 2)

    row_spec = pl.BlockSpec((BQ, E), lambda i: (i, 0))
    full = lambda a: pl.BlockSpec(a.shape, lambda i: (0,) * a.ndim)
    bo2 = b_o.reshape(1, E)

    j = np.arange(E)
    perm = (j // D) * D + np.where(j % 2 == 0, (j % D) // 2,
                                   D // 2 + (j % D) // 2)
    W_vp = W_v[:, perm]
    bv2 = b_v[perm].reshape(1, E)

    W_sox = W_so.reshape(E, H * P, 2)[:, :, 0]
    W_soy = W_so.reshape(E, H * P, 2)[:, :, 1]
    b_sox = b_so.reshape(1, H * P, 2)[:, :, 0]
    b_soy = b_so.reshape(1, H * P, 2)[:, :, 1]
    baw2 = b_aw.reshape(1, H * P)
    Rm = jnp.asarray(np.kron(np.eye(H * P, dtype=np.float32),
                             np.ones((1, 4), np.float32)))
    Sm = jnp.asarray(np.kron(np.eye(H, dtype=np.float32),
                             np.ones((P, P), np.float32)))
    lane = np.arange(H * K)
    consts = np.zeros((8, H * K), np.float32)
    consts[0] = lane % 4 % 2
    consts[1] = lane % 4 // 2
    consts[2] = lane // K
    Cm = jnp.asarray(consts)

    vproj, idxp, wp = pl.pallas_call(
        _proj_kernel,
        grid=(NBLK,),
        in_specs=[row_spec, row_spec, pl.BlockSpec((BQ, 2), lambda i: (i, 0)),
                  full(W_vp), full(bv2), full(W_sox), full(b_sox),
                  full(W_soy), full(b_soy), full(W_aw), full(baw2),
                  full(Rm), full(Sm), full(Cm)],
        out_specs=[row_spec,
                   pl.BlockSpec((BQ, H * K), lambda i: (i, 0)),
                   pl.BlockSpec((BQ, H * K), lambda i: (i, 0))],
        out_shape=[jax.ShapeDtypeStruct((B * NQ, E), jnp.bfloat16),
                   jax.ShapeDtypeStruct((B * NQ, H * K), jnp.int32),
                   jax.ShapeDtypeStruct((B * NQ, H * K), jnp.float32)],
    )(q2, v2, rp2, W_vp, bv2, W_sox, b_sox, W_soy, b_soy, W_aw, baw2,
      Rm, Sm, Cm)

    table = vproj.reshape(ITEMS, D)

    sampled = pl.kernel(
        _sc_gather_kernel,
        out_type=jax.ShapeDtypeStruct((B * NQ, E), jnp.float32),
        mesh=plsc.VectorSubcoreMesh(core_axis_name="c", subcore_axis_name="s",
                                    num_cores=2, num_subcores=16),
        scratch_types=[
            pltpu.VMEM((QC, H * K), jnp.int32),
            pltpu.VMEM((QC, H * K), jnp.int32),
            pltpu.VMEM((QC, H * K), jnp.float32),
            pltpu.VMEM((QC, H * K), jnp.float32),
            pltpu.VMEM((QC * H * K, D), jnp.bfloat16),
            pltpu.VMEM((QC * H * K, D), jnp.bfloat16),
            pltpu.VMEM((QC, E), jnp.float32),
            pltpu.VMEM((QC, E), jnp.float32),
            pltpu.SemaphoreType.DMA,
            pltpu.SemaphoreType.DMA,
            pltpu.SemaphoreType.DMA,
            pltpu.SemaphoreType.DMA,
            pltpu.SemaphoreType.DMA,
            pltpu.SemaphoreType.DMA,
        ],
        compiler_params=pltpu.CompilerParams(use_tc_tiling_on_sc=False,
                                             needs_layout_passes=False),
    )(table, idxp, wp)

    out = pl.pallas_call(
        _out_kernel,
        grid=(NBLK,),
        in_specs=[row_spec, row_spec, full(W_o), full(bo2)],
        out_specs=row_spec,
        out_shape=jax.ShapeDtypeStruct((B * NQ, E), jnp.float32),
    )(sampled, q2, W_o, bo2)

    return out.reshape(B, NQ, E)


# speedup vs baseline: 1.0730x; 1.0730x over previous
import functools
import math

import jax
import jax.numpy as jnp
import numpy as np
from jax import lax
from jax.experimental import pallas as pl
from jax.experimental.pallas import tpu as pltpu
from jax.experimental.pallas import tpu_sc as plsc

B = 2
NQ = 10000
E = 256
H = 8
D = 32
P = 4
GH = 100
GW = 100
HW = GH * GW
ITEMS = B * NQ * H
K = 4 * P

NTILES = 32
IPT = ITEMS // NTILES
C = 40
NCHUNK = IPT // C
GSUB = (C * K) // 128

BQ = 2000
NBLK = (B * NQ) // BQ


def _proj_kernel(q_ref, v_ref, rp_ref, wvlo_ref, bvlo_ref, wvhi_ref, bvhi_ref,
                 wsox_ref, bsox_ref, wsoy_ref, bsoy_ref, waw_ref, baw_ref,
                 r_ref, s_ref, c_ref, vout_ref, idx_ref, w_ref):
    f32 = jnp.float32
    q = q_ref[...]
    v = v_ref[...]
    vlo = jnp.dot(v, wvlo_ref[...], preferred_element_type=f32) + bvlo_ref[...]
    vhi = jnp.dot(v, wvhi_ref[...], preferred_element_type=f32) + bvhi_ref[...]
    lo16 = lax.bitcast_convert_type(vlo.astype(jnp.bfloat16), jnp.uint16)
    hi16 = lax.bitcast_convert_type(vhi.astype(jnp.bfloat16), jnp.uint16)
    vout_ref[...] = ((hi16.astype(jnp.int32) << 16)
                     | lo16.astype(jnp.int32))
    sox = jnp.dot(q, wsox_ref[...], preferred_element_type=f32) + bsox_ref[...]
    soy = jnp.dot(q, wsoy_ref[...], preferred_element_type=f32) + bsoy_ref[...]
    awl = jnp.dot(q, waw_ref[...], preferred_element_type=f32) + baw_ref[...]
    e = jnp.exp(awl - jnp.max(awl, axis=-1, keepdims=True))
    awn = e / jnp.dot(e, s_ref[...], preferred_element_type=f32)
    x = (rp_ref[:, 0:1] + sox / f32(GW)) * f32(GW) - f32(0.5)
    y = (rp_ref[:, 1:2] + soy / f32(GH)) * f32(GH) - f32(0.5)
    x0 = jnp.floor(x)
    y0 = jnp.floor(y)
    lx = x - x0
    ly = y - y0
    r = r_ref[...]
    X0 = jnp.dot(x0, r, preferred_element_type=f32)
    Y0 = jnp.dot(y0, r, preferred_element_type=f32)
    LX = jnp.dot(lx, r, preferred_element_type=f32)
    LY = jnp.dot(ly, r, preferred_element_type=f32)
    AW = jnp.dot(awn, r, preferred_element_type=f32)
    dx = c_ref[0:1, :]
    dy = c_ref[1:2, :]
    hl = c_ref[2:3, :]
    xi = X0 + dx
    yi = Y0 + dy
    fx = dx * LX + (f32(1.0) - dx) * (f32(1.0) - LX)
    fy = dy * LY + (f32(1.0) - dy) * (f32(1.0) - LY)
    m = ((xi >= f32(0.0)) & (xi <= f32(GW - 1)) & (yi >= f32(0.0))
         & (yi <= f32(GH - 1))).astype(f32)
    cell = (jnp.clip(yi, f32(0.0), f32(GH - 1)) * f32(GW)
            + jnp.clip(xi, f32(0.0), f32(GW - 1)))
    boff = lax.convert_element_type(
        (pl.program_id(0) // (NQ // BQ)) * (HW * H), f32)
    idx_ref[...] = (cell * f32(H) + (boff + hl)).astype(jnp.int32)
    w_ref[...] = AW * fx * fy * m


def _out_kernel(s_ref, q_ref, wo_ref, bo_ref, out_ref):
    out_ref[...] = (jnp.dot(s_ref[...], wo_ref[...],
                            preferred_element_type=jnp.float32)
                    + bo_ref[...] + q_ref[...])


QC = 5
QPT = (B * NQ) // NTILES


def _bcast_lane(v, k):
    return lax.gather(
        v, jnp.full((16, 1), k, jnp.int32),
        lax.GatherDimensionNumbers(offset_dims=(), collapsed_slice_dims=(0,),
                                   start_index_map=(0,)),
        slice_sizes=(1,), mode=lax.GatherScatterMode.PROMISE_IN_BOUNDS)


NCH = QPT // QC


def _sc_gather_kernel(table, idxp, wp, out,
                      idx_v0, idx_v1, w_v0, w_v1, rows_v0, rows_v1,
                      out_v0, out_v1,
                      sem_in0, sem_in1, sem_g0, sem_g1, sem_out0, sem_out1):
    wid = lax.axis_index("s") * 2 + lax.axis_index("c")
    base = wid * QPT
    bufs = ((idx_v0, w_v0, rows_v0, out_v0, sem_in0, sem_g0, sem_out0),
            (idx_v1, w_v1, rows_v1, out_v1, sem_in1, sem_g1, sem_out1))

    def in_cps(g, buf):
        idx_v, w_v, _, _, sem_in, _, _ = bufs[buf]
        q0 = base + g * QC
        return (pltpu.make_async_copy(idxp.at[pl.ds(q0, QC), :], idx_v, sem_in),
                pltpu.make_async_copy(wp.at[pl.ds(q0, QC), :], w_v, sem_in))

    def g_cps(buf):
        idx_v, _, rows_v, _, _, sem_g, _ = bufs[buf]
        return [pltpu.make_async_copy(table.at[idx_v.at[j, :]],
                                      rows_v.at[pl.ds(j * 128, 128), :], sem_g)
                for j in range(QC)]

    def out_cp(g, buf):
        _, _, _, out_v, _, _, sem_out = bufs[buf]
        q0 = base + g * QC
        return pltpu.make_async_copy(out_v, out.at[pl.ds(q0, QC), :], sem_out)

    def compute(buf):
        _, w_v, rows_v, out_v, _, _, _ = bufs[buf]

        def query(ql, c2):
            for h in range(H):
                acc0 = jnp.zeros((16,), jnp.float32)
                acc1 = jnp.zeros((16,), jnp.float32)
                wv = w_v[ql, pl.ds(h * K, K)]
                for k in range(K):
                    r = ql * 128 + h * K + k
                    u = rows_v[r, :]
                    lo = plsc.bitcast(u << 16, jnp.float32)
                    hi = plsc.bitcast(u & jnp.int32(-65536), jnp.float32)
                    wk = _bcast_lane(wv, k)
                    acc0 = acc0 + wk * lo
                    acc1 = acc1 + wk * hi
                out_v[ql, pl.ds(h * D, 16)] = acc0
                out_v[ql, pl.ds(h * D + 16, 16)] = acc1
            return c2

        lax.fori_loop(0, QC, query, 0, unroll=False)

    def step(g, buf, i):
        for cp in g_cps(buf):
            cp.wait()
        for cp in in_cps(g + 1, 1 - buf):
            cp.wait()
        for cp in g_cps(1 - buf):
            cp.start()
        @pl.when(i >= 1)
        def _drain():
            out_cp(g - 2, buf).wait()

        compute(buf)
        out_cp(g, buf).start()

        @pl.when(g + 2 < NCH)
        def _prefetch():
            for cp in in_cps(g + 2, buf):
                cp.start()

    for cp in in_cps(0, 0) + in_cps(1, 1):
        cp.start()
    for cp in in_cps(0, 0):
        cp.wait()
    for cp in g_cps(0):
        cp.start()

    def body(i, carry):
        step(2 * i, 0, i)
        step(2 * i + 1, 1, i)
        return carry

    lax.fori_loop(0, (NCH - 1) // 2, body, 0, unroll=False)

    g_last = NCH - 1
    for cp in g_cps(0):
        cp.wait()
    out_cp(g_last - 2, 0).wait()
    compute(0)
    out_cp(g_last, 0).start()
    out_cp(g_last - 1, 1).wait()
    out_cp(g_last, 0).wait()


def _build_idx_w(so, aw_logits, reference_points):
    so = so.reshape(B, NQ, H, P, 2)
    aw = jax.nn.softmax(aw_logits.reshape(B, NQ, H, P), axis=-1)
    rp = reference_points.reshape(B, NQ, 1, 1, 2)
    loc = rp + so / jnp.array([GW, GH], jnp.float32)
    x = loc[..., 0] * GW - 0.5
    y = loc[..., 1] * GH - 0.5
    x0 = jnp.floor(x)
    y0 = jnp.floor(y)
    lx = x - x0
    ly = y - y0
    x0i = x0.astype(jnp.int32)
    y0i = y0.astype(jnp.int32)
    b_off = (jnp.arange(B, dtype=jnp.int32) * HW).reshape(B, 1, 1, 1)
    h_off = jnp.arange(H, dtype=jnp.int32).reshape(1, 1, H, 1)
    idxs, ws = [], []
    for dy, dx, cw in ((0, 0, (1 - ly) * (1 - lx)), (0, 1, (1 - ly) * lx),
                       (1, 0, ly * (1 - lx)), (1, 1, ly * lx)):
        yi = y0i + dy
        xi = x0i + dx
        m = (yi >= 0) & (yi < GH) & (xi >= 0) & (xi < GW)
        cell = jnp.clip(yi, 0, GH - 1) * GW + jnp.clip(xi, 0, GW - 1)
        idxs.append((b_off + cell) * H + h_off)
        ws.append(aw * cw * m.astype(jnp.float32))
    idx = jnp.stack(idxs, axis=-1)
    w = jnp.stack(ws, axis=-1)
    return idx.reshape(-1), w.reshape(-1)


def kernel(query, value, reference_points, spatial_shapes, W_so, b_so, W_aw,
           b_aw, W_v, b_v, W_o, b_o):
    q2 = query.reshape(B * NQ, E)
    v2 = value.reshape(B * NQ, E)
    rp2 = reference_points.reshape(B * NQ, 2)

    row_spec = pl.BlockSpec((BQ, E), lambda i: (i, 0))
    full = lambda a: pl.BlockSpec(a.shape, lambda i: (0,) * a.ndim)
    bo2 = b_o.reshape(1, E)

    W_vg = W_v.reshape(E, H, D)
    W_vlo = W_vg[:, :, :D // 2].reshape(E, E // 2)
    W_vhi = W_vg[:, :, D // 2:].reshape(E, E // 2)
    b_vg = b_v.reshape(H, D)
    b_vlo = b_vg[:, :D // 2].reshape(1, E // 2)
    b_vhi = b_vg[:, D // 2:].reshape(1, E // 2)

    W_sox = W_so.reshape(E, H * P, 2)[:, :, 0]
    W_soy = W_so.reshape(E, H * P, 2)[:, :, 1]
    b_sox = b_so.reshape(1, H * P, 2)[:, :, 0]
    b_soy = b_so.reshape(1, H * P, 2)[:, :, 1]
    baw2 = b_aw.reshape(1, H * P)
    Rm = jnp.asarray(np.kron(np.eye(H * P, dtype=np.float32),
                             np.ones((1, 4), np.float32)))
    Sm = jnp.asarray(np.kron(np.eye(H, dtype=np.float32),
                             np.ones((P, P), np.float32)))
    lane = np.arange(H * K)
    consts = np.zeros((8, H * K), np.float32)
    consts[0] = lane % 4 % 2
    consts[1] = lane % 4 // 2
    consts[2] = lane // K
    Cm = jnp.asarray(consts)

    vproj, idxp, wp = pl.pallas_call(
        _proj_kernel,
        grid=(NBLK,),
        in_specs=[row_spec, row_spec, pl.BlockSpec((BQ, 2), lambda i: (i, 0)),
                  full(W_vlo), full(b_vlo), full(W_vhi), full(b_vhi),
                  full(W_sox), full(b_sox),
                  full(W_soy), full(b_soy), full(W_aw), full(baw2),
                  full(Rm), full(Sm), full(Cm)],
        out_specs=[pl.BlockSpec((BQ, E // 2), lambda i: (i, 0)),
                   pl.BlockSpec((BQ, H * K), lambda i: (i, 0)),
                   pl.BlockSpec((BQ, H * K), lambda i: (i, 0))],
        out_shape=[jax.ShapeDtypeStruct((B * NQ, E // 2), jnp.int32),
                   jax.ShapeDtypeStruct((B * NQ, H * K), jnp.int32),
                   jax.ShapeDtypeStruct((B * NQ, H * K), jnp.float32)],
    )(q2, v2, rp2, W_vlo, b_vlo, W_vhi, b_vhi, W_sox, b_sox, W_soy, b_soy,
      W_aw, baw2, Rm, Sm, Cm)

    table = vproj.reshape(ITEMS, D // 2)

    sampled = pl.kernel(
        _sc_gather_kernel,
        out_type=jax.ShapeDtypeStruct((B * NQ, E), jnp.float32),
        mesh=plsc.VectorSubcoreMesh(core_axis_name="c", subcore_axis_name="s",
                                    num_cores=2, num_subcores=16),
        scratch_types=[
            pltpu.VMEM((QC, H * K), jnp.int32),
            pltpu.VMEM((QC, H * K), jnp.int32),
            pltpu.VMEM((QC, H * K), jnp.float32),
            pltpu.VMEM((QC, H * K), jnp.float32),
            pltpu.VMEM((QC * H * K, D // 2), jnp.int32),
            pltpu.VMEM((QC * H * K, D // 2), jnp.int32),
            pltpu.VMEM((QC, E), jnp.float32),
            pltpu.VMEM((QC, E), jnp.float32),
            pltpu.SemaphoreType.DMA,
            pltpu.SemaphoreType.DMA,
            pltpu.SemaphoreType.DMA,
            pltpu.SemaphoreType.DMA,
            pltpu.SemaphoreType.DMA,
            pltpu.SemaphoreType.DMA,
        ],
        compiler_params=pltpu.CompilerParams(use_tc_tiling_on_sc=False,
                                             needs_layout_passes=False),
    )(table, idxp, wp)

    out = pl.pallas_call(
        _out_kernel,
        grid=(NBLK,),
        in_specs=[row_spec, row_spec, full(W_o), full(bo2)],
        out_specs=row_spec,
        out_shape=jax.ShapeDtypeStruct((B * NQ, E), jnp.float32),
    )(sampled, q2, W_o, bo2)

    return out.reshape(B, NQ, E)


# speedup vs baseline: 1.2795x; 1.1924x over previous
import functools
import math

import jax
import jax.numpy as jnp
import numpy as np
from jax import lax
from jax.experimental import pallas as pl
from jax.experimental.pallas import tpu as pltpu
from jax.experimental.pallas import tpu_sc as plsc

B = 2
NQ = 10000
E = 256
H = 8
D = 32
P = 4
GH = 100
GW = 100
HW = GH * GW
ITEMS = B * NQ * H
K = 4 * P

NTILES = 32
IPT = ITEMS // NTILES
C = 40
NCHUNK = IPT // C
GSUB = (C * K) // 128

BQ = 2000
NBLK = (B * NQ) // BQ


def _proj_kernel(q_ref, v_ref, rp_ref, wvlo_ref, bvlo_ref, wvhi_ref, bvhi_ref,
                 wsox_ref, bsox_ref, wsoy_ref, bsoy_ref, waw_ref, baw_ref,
                 r_ref, s_ref, c_ref, vout_ref, idx_ref, w_ref):
    f32 = jnp.float32
    q = q_ref[...]
    v = v_ref[...]
    vlo = jnp.dot(v, wvlo_ref[...], preferred_element_type=f32) + bvlo_ref[...]
    vhi = jnp.dot(v, wvhi_ref[...], preferred_element_type=f32) + bvhi_ref[...]
    lo16 = lax.bitcast_convert_type(vlo.astype(jnp.bfloat16), jnp.uint16)
    hi16 = lax.bitcast_convert_type(vhi.astype(jnp.bfloat16), jnp.uint16)
    vout_ref[...] = ((hi16.astype(jnp.int32) << 16)
                     | lo16.astype(jnp.int32))
    sox = jnp.dot(q, wsox_ref[...], preferred_element_type=f32) + bsox_ref[...]
    soy = jnp.dot(q, wsoy_ref[...], preferred_element_type=f32) + bsoy_ref[...]
    awl = jnp.dot(q, waw_ref[...], preferred_element_type=f32) + baw_ref[...]
    e = jnp.exp(awl - jnp.max(awl, axis=-1, keepdims=True))
    awn = e / jnp.dot(e, s_ref[...], preferred_element_type=f32)
    x = (rp_ref[:, 0:1] + sox / f32(GW)) * f32(GW) - f32(0.5)
    y = (rp_ref[:, 1:2] + soy / f32(GH)) * f32(GH) - f32(0.5)
    x0 = jnp.floor(x)
    y0 = jnp.floor(y)
    lx = x - x0
    ly = y - y0
    r = r_ref[...]
    X0 = jnp.dot(x0, r, preferred_element_type=f32)
    Y0 = jnp.dot(y0, r, preferred_element_type=f32)
    LX = jnp.dot(lx, r, preferred_element_type=f32)
    LY = jnp.dot(ly, r, preferred_element_type=f32)
    AW = jnp.dot(awn, r, preferred_element_type=f32)
    dx = c_ref[0:1, :]
    dy = c_ref[1:2, :]
    hl = c_ref[2:3, :]
    xi = X0 + dx
    yi = Y0 + dy
    fx = dx * LX + (f32(1.0) - dx) * (f32(1.0) - LX)
    fy = dy * LY + (f32(1.0) - dy) * (f32(1.0) - LY)
    m = ((xi >= f32(0.0)) & (xi <= f32(GW - 1)) & (yi >= f32(0.0))
         & (yi <= f32(GH - 1))).astype(f32)
    cell = (jnp.clip(yi, f32(0.0), f32(GH - 1)) * f32(GW)
            + jnp.clip(xi, f32(0.0), f32(GW - 1)))
    boff = lax.convert_element_type(
        (pl.program_id(0) // (NQ // BQ)) * (HW * H), f32)
    idx_ref[...] = (cell * f32(H) + (boff + hl)).astype(jnp.int32)
    w_ref[...] = AW * fx * fy * m


def _out_kernel(s_ref, q_ref, wo_ref, bo_ref, out_ref):
    out_ref[...] = (jnp.dot(s_ref[...], wo_ref[...],
                            preferred_element_type=jnp.float32)
                    + bo_ref[...] + q_ref[...])


QC = 25
QPT = (B * NQ) // NTILES


def _bcast_lane(v, k):
    return lax.gather(
        v, jnp.full((16, 1), k, jnp.int32),
        lax.GatherDimensionNumbers(offset_dims=(), collapsed_slice_dims=(0,),
                                   start_index_map=(0,)),
        slice_sizes=(1,), mode=lax.GatherScatterMode.PROMISE_IN_BOUNDS)


NCH = QPT // QC


def _sc_gather_kernel(table, idxp, wp, out,
                      idx_v0, idx_v1, w_v0, w_v1, rows_v0, rows_v1, out_v,
                      sem_in0, sem_in1, sem_g0, sem_g1, sem_out):
    wid = lax.axis_index("s") * 2 + lax.axis_index("c")
    base = wid * QPT
    bufs = ((idx_v0, w_v0, rows_v0, sem_in0, sem_g0),
            (idx_v1, w_v1, rows_v1, sem_in1, sem_g1))

    def in_cps(g, buf):
        idx_v, w_v, _, sem_in, _ = bufs[buf]
        q0 = base + g * QC
        return (pltpu.make_async_copy(idxp.at[pl.ds(q0, QC), :], idx_v, sem_in),
                pltpu.make_async_copy(wp.at[pl.ds(q0, QC), :], w_v, sem_in))

    def g_cps(buf):
        idx_v, _, rows_v, _, sem_g = bufs[buf]
        return [pltpu.make_async_copy(table.at[idx_v.at[j, :]],
                                      rows_v.at[pl.ds(j * 128, 128), :], sem_g)
                for j in range(QC)]

    def out_cp(g):
        q0 = base + g * QC
        return pltpu.make_async_copy(out_v, out.at[pl.ds(q0, QC), :], sem_out)

    def compute(buf):
        _, w_v, rows_v, _, _ = bufs[buf]

        def query(ql, c2):
            for h in range(H):
                acc0 = jnp.zeros((16,), jnp.float32)
                acc1 = jnp.zeros((16,), jnp.float32)
                wv = w_v[ql, pl.ds(h * K, K)]
                for k in range(K):
                    r = ql * 128 + h * K + k
                    u = rows_v[r, :]
                    lo = plsc.bitcast(u << 16, jnp.float32)
                    hi = plsc.bitcast(u & jnp.int32(-65536), jnp.float32)
                    wk = _bcast_lane(wv, k)
                    acc0 = acc0 + wk * lo
                    acc1 = acc1 + wk * hi
                out_v[ql, pl.ds(h * D, 16)] = acc0
                out_v[ql, pl.ds(h * D + 16, 16)] = acc1
            return c2

        lax.fori_loop(0, QC, query, 0, unroll=False)

    def step(g, buf, i):
        for cp in g_cps(buf):
            cp.wait()
        for cp in in_cps(g + 1, 1 - buf):
            cp.wait()
        for cp in g_cps(1 - buf):
            cp.start()
        if buf == 1:
            out_cp(g - 1).wait()
        else:
            @pl.when(i >= 1)
            def _drain():
                out_cp(g - 1).wait()

        compute(buf)
        out_cp(g).start()

        @pl.when(g + 2 < NCH)
        def _prefetch():
            for cp in in_cps(g + 2, buf):
                cp.start()

    for cp in in_cps(0, 0) + in_cps(1, 1):
        cp.start()
    for cp in in_cps(0, 0):
        cp.wait()
    for cp in g_cps(0):
        cp.start()

    def body(i, carry):
        step(2 * i, 0, i)
        step(2 * i + 1, 1, i)
        return carry

    lax.fori_loop(0, (NCH - 1) // 2, body, 0, unroll=False)

    g_last = NCH - 1
    for cp in g_cps(0):
        cp.wait()
    out_cp(g_last - 1).wait()
    compute(0)
    out_cp(g_last).start()
    out_cp(g_last).wait()


def _build_idx_w(so, aw_logits, reference_points):
    so = so.reshape(B, NQ, H, P, 2)
    aw = jax.nn.softmax(aw_logits.reshape(B, NQ, H, P), axis=-1)
    rp = reference_points.reshape(B, NQ, 1, 1, 2)
    loc = rp + so / jnp.array([GW, GH], jnp.float32)
    x = loc[..., 0] * GW - 0.5
    y = loc[..., 1] * GH - 0.5
    x0 = jnp.floor(x)
    y0 = jnp.floor(y)
    lx = x - x0
    ly = y - y0
    x0i = x0.astype(jnp.int32)
    y0i = y0.astype(jnp.int32)
    b_off = (jnp.arange(B, dtype=jnp.int32) * HW).reshape(B, 1, 1, 1)
    h_off = jnp.arange(H, dtype=jnp.int32).reshape(1, 1, H, 1)
    idxs, ws = [], []
    for dy, dx, cw in ((0, 0, (1 - ly) * (1 - lx)), (0, 1, (1 - ly) * lx),
                       (1, 0, ly * (1 - lx)), (1, 1, ly * lx)):
        yi = y0i + dy
        xi = x0i + dx
        m = (yi >= 0) & (yi < GH) & (xi >= 0) & (xi < GW)
        cell = jnp.clip(yi, 0, GH - 1) * GW + jnp.clip(xi, 0, GW - 1)
        idxs.append((b_off + cell) * H + h_off)
        ws.append(aw * cw * m.astype(jnp.float32))
    idx = jnp.stack(idxs, axis=-1)
    w = jnp.stack(ws, axis=-1)
    return idx.reshape(-1), w.reshape(-1)


def kernel(query, value, reference_points, spatial_shapes, W_so, b_so, W_aw,
           b_aw, W_v, b_v, W_o, b_o):
    q2 = query.reshape(B * NQ, E)
    v2 = value.reshape(B * NQ, E)
    rp2 = reference_points.reshape(B * NQ, 2)

    row_spec = pl.BlockSpec((BQ, E), lambda i: (i, 0))
    full = lambda a: pl.BlockSpec(a.shape, lambda i: (0,) * a.ndim)
    bo2 = b_o.reshape(1, E)

    W_vg = W_v.reshape(E, H, D)
    W_vlo = W_vg[:, :, :D // 2].reshape(E, E // 2)
    W_vhi = W_vg[:, :, D // 2:].reshape(E, E // 2)
    b_vg = b_v.reshape(H, D)
    b_vlo = b_vg[:, :D // 2].reshape(1, E // 2)
    b_vhi = b_vg[:, D // 2:].reshape(1, E // 2)

    W_sox = W_so.reshape(E, H * P, 2)[:, :, 0]
    W_soy = W_so.reshape(E, H * P, 2)[:, :, 1]
    b_sox = b_so.reshape(1, H * P, 2)[:, :, 0]
    b_soy = b_so.reshape(1, H * P, 2)[:, :, 1]
    baw2 = b_aw.reshape(1, H * P)
    Rm = jnp.asarray(np.kron(np.eye(H * P, dtype=np.float32),
                             np.ones((1, 4), np.float32)))
    Sm = jnp.asarray(np.kron(np.eye(H, dtype=np.float32),
                             np.ones((P, P), np.float32)))
    lane = np.arange(H * K)
    consts = np.zeros((8, H * K), np.float32)
    consts[0] = lane % 4 % 2
    consts[1] = lane % 4 // 2
    consts[2] = lane // K
    Cm = jnp.asarray(consts)

    vproj, idxp, wp = pl.pallas_call(
        _proj_kernel,
        grid=(NBLK,),
        in_specs=[row_spec, row_spec, pl.BlockSpec((BQ, 2), lambda i: (i, 0)),
                  full(W_vlo), full(b_vlo), full(W_vhi), full(b_vhi),
                  full(W_sox), full(b_sox),
                  full(W_soy), full(b_soy), full(W_aw), full(baw2),
                  full(Rm), full(Sm), full(Cm)],
        out_specs=[pl.BlockSpec((BQ, E // 2), lambda i: (i, 0)),
                   pl.BlockSpec((BQ, H * K), lambda i: (i, 0)),
                   pl.BlockSpec((BQ, H * K), lambda i: (i, 0))],
        out_shape=[jax.ShapeDtypeStruct((B * NQ, E // 2), jnp.int32),
                   jax.ShapeDtypeStruct((B * NQ, H * K), jnp.int32),
                   jax.ShapeDtypeStruct((B * NQ, H * K), jnp.float32)],
    )(q2, v2, rp2, W_vlo, b_vlo, W_vhi, b_vhi, W_sox, b_sox, W_soy, b_soy,
      W_aw, baw2, Rm, Sm, Cm)

    table = vproj.reshape(ITEMS, D // 2)

    sampled = pl.kernel(
        _sc_gather_kernel,
        out_type=jax.ShapeDtypeStruct((B * NQ, E), jnp.float32),
        mesh=plsc.VectorSubcoreMesh(core_axis_name="c", subcore_axis_name="s",
                                    num_cores=2, num_subcores=16),
        scratch_types=[
            pltpu.VMEM((QC, H * K), jnp.int32),
            pltpu.VMEM((QC, H * K), jnp.int32),
            pltpu.VMEM((QC, H * K), jnp.float32),
            pltpu.VMEM((QC, H * K), jnp.float32),
            pltpu.VMEM((QC * H * K, D // 2), jnp.int32),
            pltpu.VMEM((QC * H * K, D // 2), jnp.int32),
            pltpu.VMEM((QC, E), jnp.float32),
            pltpu.SemaphoreType.DMA,
            pltpu.SemaphoreType.DMA,
            pltpu.SemaphoreType.DMA,
            pltpu.SemaphoreType.DMA,
            pltpu.SemaphoreType.DMA,
        ],
        compiler_params=pltpu.CompilerParams(use_tc_tiling_on_sc=False,
                                             needs_layout_passes=False),
    )(table, idxp, wp)

    out = pl.pallas_call(
        _out_kernel,
        grid=(NBLK,),
        in_specs=[row_spec, row_spec, full(W_o), full(bo2)],
        out_specs=row_spec,
        out_shape=jax.ShapeDtypeStruct((B * NQ, E), jnp.float32),
    )(sampled, q2, W_o, bo2)

    return out.reshape(B, NQ, E)
